# baseline (speedup 1.0000x reference)
import functools

import jax
import jax.numpy as jnp
from jax.experimental import pallas as pl
from jax.experimental.pallas import tpu as pltpu


def _conv_nchw_kernel(x_ref, w_ref, b_ref, o_ref, *, h, w, kk):
    cout = o_ref.shape[1]
    m = h * w
    r = kk // 2

    xb = x_ref[0].astype(jnp.bfloat16)

    pos = jax.lax.broadcasted_iota(jnp.int32, (1, m), 1)
    col = jax.lax.rem(pos, w)
    row = jax.lax.div(pos, w)

    acc = jnp.zeros((cout, m), jnp.float32)
    for dy in range(kk):
        for dx in range(kk):
            dr, dc = dy - r, dx - r
            s = dr * w + dc
            xs = xb if s == 0 else jnp.roll(xb, -s, axis=1)
            valid = None
            if dc < 0:
                valid = col >= -dc
            elif dc > 0:
                valid = col < w - dc
            if dr < 0:
                v = row >= -dr
                valid = v if valid is None else jnp.logical_and(valid, v)
            elif dr > 0:
                v = row < h - dr
                valid = v if valid is None else jnp.logical_and(valid, v)
            if valid is not None:
                xs = jnp.where(valid, xs, jnp.bfloat16(0.0))
            acc += jnp.dot(w_ref[dy * kk + dx], xs,
                           preferred_element_type=jnp.float32)

    y = acc + b_ref[...]
    e = jnp.exp(-jnp.abs(y))
    sig = pl.reciprocal(1.0 + e, approx=True)
    sig = jnp.where(y >= 0.0, sig, 1.0 - sig)
    o_ref[0] = (y * sig).astype(jnp.bfloat16).astype(jnp.float32)


@functools.partial(jax.jit, static_argnames=("kernel_size", "eps"))
def _conv_bn_swish_nchw(x_nchw, weight, gamma, beta, running_mean,
                        running_var, *, kernel_size, eps=1e-5):
    n, cin, h, w = x_nchw.shape
    cout = weight.shape[0]
    kk = kernel_size
    m = h * w

    inv_std = gamma.astype(jnp.float32) / jnp.sqrt(
        running_var.astype(jnp.float32) + eps)
    bias = beta.astype(jnp.float32) - running_mean.astype(jnp.float32) * inv_std

    w_taps = jnp.transpose(weight.astype(jnp.float32) * inv_std[:, None, None, None],
                           (2, 3, 0, 1)).reshape(kk * kk, cout, cin)
    w_prep = w_taps.astype(jnp.bfloat16)
    b_prep = bias.reshape(cout, 1)

    x_flat = x_nchw.reshape(n, cin, m)

    kern = functools.partial(_conv_nchw_kernel, h=h, w=w, kk=kk)

    out = pl.pallas_call(
        kern,
        out_shape=jax.ShapeDtypeStruct((n, cout, m), jnp.float32),
        grid=(n,),
        in_specs=[
            pl.BlockSpec((1, cin, m), lambda i: (i, 0, 0)),
            pl.BlockSpec((kk * kk, cout, cin), lambda i: (0, 0, 0)),
            pl.BlockSpec((cout, 1), lambda i: (0, 0)),
        ],
        out_specs=pl.BlockSpec((1, cout, m), lambda i: (i, 0, 0)),
        compiler_params=pltpu.CompilerParams(
            dimension_semantics=("parallel",),
            vmem_limit_bytes=64 << 20,
        ),
        cost_estimate=pl.CostEstimate(
            flops=2 * n * m * kk * kk * cin * cout,
            transcendentals=n * m * cout,
            bytes_accessed=n * cin * m * 4 + n * cout * m * 4
            + kk * kk * cin * cout * 2),
    )(x_flat, w_prep, b_prep)

    return out.reshape(n, cout, h, w)


def kernel(x_nchw, weight, gamma, beta, running_mean, running_var):
    return _conv_bn_swish_nchw(x_nchw, weight, gamma, beta, running_mean,
                               running_var, kernel_size=3)
